# baseline (speedup 1.0000x reference)
import functools
from math import sqrt

import jax
import jax.numpy as jnp
from jax import lax
from jax.experimental import pallas as pl
from jax.experimental.pallas import tpu as pltpu
from jax.experimental.pallas import tpu_sc as plsc

_N = 10000
_E = 320000
_D = 128
_ED = 16
_H = 4
_AD = 16
_OD = 64
_NP = 10240
_NT = 32
_C = 96
_NCH = 106
_EPT = _C * _NCH
_EPAD = _EPT * _NT
_EALLOC = _EPAD + _C
_RPT = _NP // 16
_BLK = 1280
_GRID = _NP // _BLK
_AW = 80


def _proj_body(x_ref, w_ref, b_ref, gafw, gbfw, gabw, gbbw, r2):
    y = jnp.dot(x_ref[...], w_ref[...], preferred_element_type=jnp.float32)
    y = y + b_ref[...]
    gafw[...] = y[:, 0:128]
    gbfw[...] = y[:, 128:256]
    gabw[...] = y[:, 256:384]
    gbbw[...] = y[:, 384:512]
    r2[...] = y[:, 512:640]


_proj_call = pl.pallas_call(
    _proj_body,
    grid=(_GRID,),
    in_specs=[
        pl.BlockSpec((_BLK, _D), lambda i: (i, 0)),
        pl.BlockSpec((_D, 640), lambda i: (0, 0)),
        pl.BlockSpec((1, 640), lambda i: (0, 0)),
    ],
    out_specs=[pl.BlockSpec((_BLK, _D), lambda i: (i, 0))] * 5,
    out_shape=[jax.ShapeDtypeStruct((_NP, _D), jnp.float32)] * 5,
)


def _edge_body(ga_fw, gb_fw, ga_bw, gb_bw, ea_p, sd_p, zrows, out,
               accs, idx0, idx1, bufa0, bufb0, bufe0, bufa1, bufb1, bufe1,
               msg, sa0, sb0, se0, sa1, sb1, se1):
    c = lax.axis_index("c")
    s = lax.axis_index("s")
    wid = s * 2 + c
    r0 = s * _RPT

    lane = lax.iota(jnp.int32, 16)
    ix1 = lane ^ 1
    ix2 = lane ^ 2
    pack_ix = [((lane - 4 * h) & 3) * 4 for h in range(_H)]
    pack_msk = [(lane >> 2) == h for h in range(1, _H)]
    bcast_ix = [lane * 0 + 4 * h for h in range(_H)]
    wpk_ix = (lane & 3) * 4
    base_t = wid * _EPT
    sets = ((idx0, bufa0, bufb0, bufe0, sa0, sb0, se0),
            (idx1, bufa1, bufb1, bufe1, sa1, sb1, se1))

    def run_dir(d, ga_hbm, gb_hbm):
        agg_row = d
        oth_row = 1 - d
        pltpu.sync_copy(zrows.at[pl.ds(r0, _RPT)], accs.at[pl.ds(r0, _RPT)])
        plsc.subcore_barrier()

        def issue(g, st):
            idx, bufa, bufb, bufe, sa, sb, se = sets[st]
            base = base_t + g * _C
            pltpu.sync_copy(sd_p.at[:, pl.ds(base, _C)], idx)
            pltpu.async_copy(ga_hbm.at[idx.at[agg_row]], bufa, sa)
            pltpu.async_copy(gb_hbm.at[idx.at[oth_row]], bufb, sb)
            pltpu.async_copy(ea_p.at[pl.ds(base, _C)], bufe, se)

        def wait(st):
            idx, bufa, bufb, bufe, sa, sb, se = sets[st]
            pltpu.make_async_copy(ga_hbm.at[idx.at[agg_row]], bufa, sa).wait()
            pltpu.make_async_copy(gb_hbm.at[idx.at[oth_row]], bufb, sb).wait()
            pltpu.make_async_copy(ea_p.at[pl.ds(0, _C)], bufe, se).wait()

        def compute(st):
            idx, bufa, bufb, bufe, sa, sb, se = sets[st]

            def edge(i, ecarry):
                ea_v = bufe[i, :]
                pk = None
                for h in range(_H):
                    qa = bufa[i, pl.ds(h * 16, 16)]
                    th = bufa[i, pl.ds(64 + h * 16, 16)]
                    kh = bufb[i, pl.ds(h * 16, 16)]
                    p = qa * kh + ea_v * th
                    p = p + p.at[ix1].get(mode='promise_in_bounds')
                    p = p + p.at[ix2].get(mode='promise_in_bounds')
                    ph = p.at[pack_ix[h]].get(mode='promise_in_bounds')
                    pk = ph if h == 0 else jnp.where(pack_msk[h - 1], ph, pk)
                pk = pk + pk.at[ix1].get(mode='promise_in_bounds')
                pk = pk + pk.at[ix2].get(mode='promise_in_bounds')
                w = jnp.exp(pk)
                for h in range(_H):
                    vh = bufb[i, pl.ds(64 + h * 16, 16)]
                    wb = w.at[bcast_ix[h]].get(mode='promise_in_bounds')
                    msg[i, pl.ds(h * 16, 16)] = wb * vh
                msg[i, pl.ds(64, 16)] = w.at[wpk_ix].get(
                    mode='promise_in_bounds')
                return ecarry

            lax.fori_loop(0, _C, edge, 0)
            pltpu.sync_copy(msg, accs.at[idx.at[agg_row]], add=True)

        issue(0, 0)

        def pair(k, carry):
            issue(2 * k + 1, 1)
            wait(0)
            compute(0)
            issue(2 * k + 2, 0)
            wait(1)
            compute(1)
            return carry

        lax.fori_loop(0, _NCH // 2, pair, 0)
        wait(0)
        plsc.subcore_barrier()
        pltpu.sync_copy(accs.at[pl.ds(r0, _RPT)],
                        out.at[c, d, pl.ds(r0, _RPT)])
        plsc.subcore_barrier()

    run_dir(0, ga_fw, gb_fw)
    run_dir(1, ga_bw, gb_bw)


@functools.cache
def _get_edge_call():
    return pl.kernel(
        _edge_body,
        out_type=jax.ShapeDtypeStruct((2, 2, _NP, _AW), jnp.float32),
        mesh=plsc.VectorSubcoreMesh(core_axis_name="c", subcore_axis_name="s"),
        scratch_types=[
            pltpu.VMEM_SHARED((_NP, _AW), jnp.float32),
            pltpu.VMEM((2, _C), jnp.int32),
            pltpu.VMEM((2, _C), jnp.int32),
            pltpu.VMEM((_C, _D), jnp.float32),
            pltpu.VMEM((_C, _D), jnp.float32),
            pltpu.VMEM((_C, _ED), jnp.float32),
            pltpu.VMEM((_C, _D), jnp.float32),
            pltpu.VMEM((_C, _D), jnp.float32),
            pltpu.VMEM((_C, _ED), jnp.float32),
            pltpu.VMEM((_C, _AW), jnp.float32),
            pltpu.SemaphoreType.DMA,
            pltpu.SemaphoreType.DMA,
            pltpu.SemaphoreType.DMA,
            pltpu.SemaphoreType.DMA,
            pltpu.SemaphoreType.DMA,
            pltpu.SemaphoreType.DMA,
        ],
        compiler_params=pltpu.CompilerParams(needs_layout_passes=False,
                                             use_tc_tiling_on_sc=False),
    )


def _combine_body(acc_ref, r2_ref, resid_ref, gw_ref, gb_ref,
                  m1w_ref, m1b_ref, m2w_ref, m2b_ref, lng_ref, lnb_ref,
                  o_ref, *, has_resid):
    acc = acc_ref[...]
    ys = []
    for dcol in range(2):
        a = acc[0, dcol] + acc[1, dcol]
        parts = [a[:, h * 16:(h + 1) * 16] / (a[:, 64 + h:65 + h] + 1e-16)
                 for h in range(_H)]
        outv = jnp.concatenate(parts, axis=1)
        res = r2_ref[:, dcol * 64:(dcol + 1) * 64]
        gin = jnp.dot(jnp.concatenate([outv, res], axis=1),
                      gw_ref[:, dcol:dcol + 1],
                      preferred_element_type=jnp.float32)
        gate = jax.nn.sigmoid(gin + gb_ref[0, dcol])
        ys.append(gate * outv + (1.0 - gate) * res)
    ret = jnp.concatenate(ys, axis=1)
    z = (jnp.dot(ret, m1w_ref[...], preferred_element_type=jnp.float32)
         + m1b_ref[...])
    h1 = 0.5 * z * (1.0 + lax.erf(z * (1.0 / sqrt(2.0))))
    y = jnp.dot(h1, m2w_ref[...], preferred_element_type=jnp.float32)
    y = y + m2b_ref[...] + ret
    mean = jnp.mean(y, axis=1, keepdims=True)
    var = jnp.mean((y - mean) ** 2, axis=1, keepdims=True)
    y = (y - mean) / jnp.sqrt(var + 1e-5) * lng_ref[...] + lnb_ref[...]
    if has_resid:
        y = y + resid_ref[...]
    o_ref[...] = y


def _make_combine(has_resid):
    return pl.pallas_call(
        functools.partial(_combine_body, has_resid=has_resid),
        grid=(_GRID,),
        in_specs=[
            pl.BlockSpec((2, 2, _BLK, _AW), lambda i: (0, 0, i, 0)),
            pl.BlockSpec((_BLK, _D), lambda i: (i, 0)),
            pl.BlockSpec((_BLK, _D), lambda i: (i, 0)),
            pl.BlockSpec((_D, 2), lambda i: (0, 0)),
            pl.BlockSpec((1, 2), lambda i: (0, 0)),
            pl.BlockSpec((_D, 2 * _D), lambda i: (0, 0)),
            pl.BlockSpec((1, 2 * _D), lambda i: (0, 0)),
            pl.BlockSpec((2 * _D, _D), lambda i: (0, 0)),
            pl.BlockSpec((1, _D), lambda i: (0, 0)),
            pl.BlockSpec((1, _D), lambda i: (0, 0)),
            pl.BlockSpec((1, _D), lambda i: (0, 0)),
        ],
        out_specs=pl.BlockSpec((_BLK, _D), lambda i: (i, 0)),
        out_shape=jax.ShapeDtypeStruct((_NP, _D), jnp.float32),
    )


_combine_calls = (_make_combine(False), _make_combine(True))


def _prep_dir(p):
    scale = 1.0 / sqrt(_AD)
    qw = p['q_w'] * scale
    qb = p['q_b'] * scale
    kwx = p['k_w'][:_D]
    kwe = p['k_w'][_D:]
    k3 = kwe.reshape(_ED, _H, _AD)
    blocks = [k3[:, h, :].T for h in range(_H)]
    m = jax.scipy.linalg.block_diag(*blocks)
    wa = jnp.concatenate([qw, qw @ m], axis=1)
    ba = jnp.concatenate([qb, qb @ m])
    wb = jnp.concatenate([kwx, p['v_w']], axis=1)
    bb = jnp.concatenate([p['k_b'], p['v_b']])
    g1 = p['g_w'][:_OD, 0] + p['g_w'][2 * _OD:, 0]
    g2 = p['g_w'][_OD:2 * _OD, 0] - p['g_w'][2 * _OD:, 0]
    return wa, ba, wb, bb, jnp.concatenate([g1, g2]), p['g_b'][0]


def _prep_layer(lp):
    waf, baf, wbf, bbf, gvf, gbf = _prep_dir(lp['fw'])
    wab, bab, wbb, bbb, gvb, gbb = _prep_dir(lp['bw'])
    wcat = jnp.concatenate([waf, wbf, wab, wbb, lp['fw']['sl_w'],
                            lp['bw']['sl_w']], axis=1)
    bcat = jnp.concatenate([baf, bbf, bab, bbb, lp['fw']['sl_b'],
                            lp['bw']['sl_b']]).reshape(1, 640)
    gw = jnp.stack([gvf, gvb], axis=1)
    gb = jnp.stack([gbf, gbb]).reshape(1, 2)
    return dict(
        wcat=wcat, bcat=bcat, gw=gw, gb=gb,
        m1w=lp['m1_w'], m1b=lp['m1_b'].reshape(1, 2 * _D),
        m2w=lp['m2_w'], m2b=lp['m2_b'].reshape(1, _D),
        lng=lp['ln_g'].reshape(1, _D), lnb=lp['ln_b'].reshape(1, _D),
    )


def kernel(x, edge_index, edge_attr, params):
    f32 = jnp.float32
    i32 = jnp.int32
    x_p = jnp.zeros((_NP, _D), f32).at[:_N].set(x)
    sd_p = jnp.full((2, _EALLOC), _N, i32).at[:, :_E].set(
        edge_index.astype(i32)[::-1])
    ea_p = jnp.zeros((_EALLOC, _ED), f32).at[:_E].set(edge_attr)
    zrows = jnp.zeros((_NP, _AW), f32)

    outs = [x_p]
    for li, lp in enumerate(params['layers']):
        w = _prep_layer(lp)
        gafw, gbfw, gabw, gbbw, r2 = _proj_call(outs[-1], w['wcat'], w['bcat'])
        acc = _get_edge_call()(gafw, gbfw, gabw, gbbw, ea_p, sd_p, zrows)
        has_resid = li == 1
        resid = outs[1] if has_resid else outs[-1]
        y = _combine_calls[int(has_resid)](
            acc, r2, resid, w['gw'], w['gb'], w['m1w'], w['m1b'],
            w['m2w'], w['m2b'], w['lng'], w['lnb'])
        outs.append(y)
    return outs[-1][:_N]


# speedup vs baseline: 1.0689x; 1.0689x over previous
import functools
from math import sqrt

import jax
import jax.numpy as jnp
from jax import lax
from jax.experimental import pallas as pl
from jax.experimental.pallas import tpu as pltpu
from jax.experimental.pallas import tpu_sc as plsc

_N = 10000
_E = 320000
_D = 128
_ED = 16
_H = 4
_AD = 16
_OD = 64
_NP = 10240
_NT = 32
_C = 96
_NCH = 106
_EPT = _C * _NCH
_EPAD = _EPT * _NT
_EALLOC = _EPAD + _C
_RPT = _NP // 16
_BLK = 1280
_GRID = _NP // _BLK
_AW = 80


def _proj_body(x_ref, w_ref, b_ref, gafw, gbfw, gabw, gbbw, r2):
    y = jnp.dot(x_ref[...], w_ref[...], preferred_element_type=jnp.float32)
    y = y + b_ref[...]
    gafw[...] = y[:, 0:128]
    gbfw[...] = y[:, 128:256]
    gabw[...] = y[:, 256:384]
    gbbw[...] = y[:, 384:512]
    r2[...] = y[:, 512:640]


_proj_call = pl.pallas_call(
    _proj_body,
    grid=(_GRID,),
    in_specs=[
        pl.BlockSpec((_BLK, _D), lambda i: (i, 0)),
        pl.BlockSpec((_D, 640), lambda i: (0, 0)),
        pl.BlockSpec((1, 640), lambda i: (0, 0)),
    ],
    out_specs=[pl.BlockSpec((_BLK, _D), lambda i: (i, 0))] * 5,
    out_shape=[jax.ShapeDtypeStruct((_NP, _D), jnp.float32)] * 5,
)


def _edge_body(ga_fw, gb_fw, ga_bw, gb_bw, ea_p, sd_p, zrows, out,
               accs, idx0, idx1, bufa0, bufb0, bufe0, bufa1, bufb1, bufe1,
               msg, sa0, sb0, se0, sa1, sb1, se1):
    c = lax.axis_index("c")
    s = lax.axis_index("s")
    wid = s * 2 + c
    r0 = s * _RPT

    lane = lax.iota(jnp.int32, 16)
    ixs = [lane ^ 8, lane ^ 4, lane ^ 2, lane ^ 1]
    base_t = wid * _EPT
    sets = ((idx0, bufa0, bufb0, bufe0, sa0, sb0, se0),
            (idx1, bufa1, bufb1, bufe1, sa1, sb1, se1))

    def run_dir(d, ga_hbm, gb_hbm):
        agg_row = d
        oth_row = 1 - d
        pltpu.sync_copy(zrows.at[pl.ds(r0, _RPT)], accs.at[pl.ds(r0, _RPT)])
        plsc.subcore_barrier()

        def issue(g, st):
            idx, bufa, bufb, bufe, sa, sb, se = sets[st]
            base = base_t + g * _C
            pltpu.sync_copy(sd_p.at[:, pl.ds(base, _C)], idx)
            pltpu.async_copy(ga_hbm.at[idx.at[agg_row]], bufa, sa)
            pltpu.async_copy(gb_hbm.at[idx.at[oth_row]], bufb, sb)
            pltpu.async_copy(ea_p.at[pl.ds(base, _C)], bufe, se)

        def wait(st):
            idx, bufa, bufb, bufe, sa, sb, se = sets[st]
            pltpu.make_async_copy(ga_hbm.at[idx.at[agg_row]], bufa, sa).wait()
            pltpu.make_async_copy(gb_hbm.at[idx.at[oth_row]], bufb, sb).wait()
            pltpu.make_async_copy(ea_p.at[pl.ds(0, _C)], bufe, se).wait()

        def compute(st):
            idx, bufa, bufb, bufe, sa, sb, se = sets[st]

            def one_edge(i):
                ea_v = bufe[i, :]
                wbrd = []
                for h in range(_H):
                    qa = bufa[i, pl.ds(h * 16, 16)]
                    th = bufa[i, pl.ds(64 + h * 16, 16)]
                    kh = bufb[i, pl.ds(h * 16, 16)]
                    p = qa * kh + ea_v * th
                    for ix in ixs:
                        p = p + p.at[ix].get(mode='promise_in_bounds')
                    wbrd.append(jnp.exp(p))
                for h in range(_H):
                    vh = bufb[i, pl.ds(64 + h * 16, 16)]
                    msg[i, pl.ds(h * 16, 16)] = wbrd[h] * vh
                w_pack = jnp.where(lane == 0, wbrd[0], wbrd[1])
                w_pack = jnp.where(lane == 2, wbrd[2], w_pack)
                w_pack = jnp.where(lane == 3, wbrd[3], w_pack)
                w_pack = jnp.where(lane < 4, w_pack, 0.0)
                msg[i, pl.ds(64, 16)] = w_pack

            def edge(j, ecarry):
                one_edge(2 * j)
                one_edge(2 * j + 1)
                return ecarry

            lax.fori_loop(0, _C // 2, edge, 0)
            pltpu.sync_copy(msg, accs.at[idx.at[agg_row]], add=True)

        issue(0, 0)

        def pair(k, carry):
            issue(2 * k + 1, 1)
            wait(0)
            compute(0)
            issue(2 * k + 2, 0)
            wait(1)
            compute(1)
            return carry

        lax.fori_loop(0, _NCH // 2, pair, 0)
        wait(0)
        plsc.subcore_barrier()
        pltpu.sync_copy(accs.at[pl.ds(r0, _RPT)],
                        out.at[c, d, pl.ds(r0, _RPT)])
        plsc.subcore_barrier()

    run_dir(0, ga_fw, gb_fw)
    run_dir(1, ga_bw, gb_bw)


@functools.cache
def _get_edge_call():
    return pl.kernel(
        _edge_body,
        out_type=jax.ShapeDtypeStruct((2, 2, _NP, _AW), jnp.float32),
        mesh=plsc.VectorSubcoreMesh(core_axis_name="c", subcore_axis_name="s"),
        scratch_types=[
            pltpu.VMEM_SHARED((_NP, _AW), jnp.float32),
            pltpu.VMEM((2, _C), jnp.int32),
            pltpu.VMEM((2, _C), jnp.int32),
            pltpu.VMEM((_C, _D), jnp.float32),
            pltpu.VMEM((_C, _D), jnp.float32),
            pltpu.VMEM((_C, _ED), jnp.float32),
            pltpu.VMEM((_C, _D), jnp.float32),
            pltpu.VMEM((_C, _D), jnp.float32),
            pltpu.VMEM((_C, _ED), jnp.float32),
            pltpu.VMEM((_C, _AW), jnp.float32),
            pltpu.SemaphoreType.DMA,
            pltpu.SemaphoreType.DMA,
            pltpu.SemaphoreType.DMA,
            pltpu.SemaphoreType.DMA,
            pltpu.SemaphoreType.DMA,
            pltpu.SemaphoreType.DMA,
        ],
        compiler_params=pltpu.CompilerParams(needs_layout_passes=False,
                                             use_tc_tiling_on_sc=False),
    )


def _combine_body(acc_ref, r2_ref, resid_ref, gw_ref, gb_ref,
                  m1w_ref, m1b_ref, m2w_ref, m2b_ref, lng_ref, lnb_ref,
                  o_ref, *, has_resid):
    acc = acc_ref[...]
    ys = []
    for dcol in range(2):
        a = acc[0, dcol] + acc[1, dcol]
        parts = [a[:, h * 16:(h + 1) * 16] / (a[:, 64 + h:65 + h] + 1e-16)
                 for h in range(_H)]
        outv = jnp.concatenate(parts, axis=1)
        res = r2_ref[:, dcol * 64:(dcol + 1) * 64]
        gin = jnp.dot(jnp.concatenate([outv, res], axis=1),
                      gw_ref[:, dcol:dcol + 1],
                      preferred_element_type=jnp.float32)
        gate = jax.nn.sigmoid(gin + gb_ref[0, dcol])
        ys.append(gate * outv + (1.0 - gate) * res)
    ret = jnp.concatenate(ys, axis=1)
    z = (jnp.dot(ret, m1w_ref[...], preferred_element_type=jnp.float32)
         + m1b_ref[...])
    h1 = 0.5 * z * (1.0 + lax.erf(z * (1.0 / sqrt(2.0))))
    y = jnp.dot(h1, m2w_ref[...], preferred_element_type=jnp.float32)
    y = y + m2b_ref[...] + ret
    mean = jnp.mean(y, axis=1, keepdims=True)
    var = jnp.mean((y - mean) ** 2, axis=1, keepdims=True)
    y = (y - mean) / jnp.sqrt(var + 1e-5) * lng_ref[...] + lnb_ref[...]
    if has_resid:
        y = y + resid_ref[...]
    o_ref[...] = y


def _make_combine(has_resid):
    return pl.pallas_call(
        functools.partial(_combine_body, has_resid=has_resid),
        grid=(_GRID,),
        in_specs=[
            pl.BlockSpec((2, 2, _BLK, _AW), lambda i: (0, 0, i, 0)),
            pl.BlockSpec((_BLK, _D), lambda i: (i, 0)),
            pl.BlockSpec((_BLK, _D), lambda i: (i, 0)),
            pl.BlockSpec((_D, 2), lambda i: (0, 0)),
            pl.BlockSpec((1, 2), lambda i: (0, 0)),
            pl.BlockSpec((_D, 2 * _D), lambda i: (0, 0)),
            pl.BlockSpec((1, 2 * _D), lambda i: (0, 0)),
            pl.BlockSpec((2 * _D, _D), lambda i: (0, 0)),
            pl.BlockSpec((1, _D), lambda i: (0, 0)),
            pl.BlockSpec((1, _D), lambda i: (0, 0)),
            pl.BlockSpec((1, _D), lambda i: (0, 0)),
        ],
        out_specs=pl.BlockSpec((_BLK, _D), lambda i: (i, 0)),
        out_shape=jax.ShapeDtypeStruct((_NP, _D), jnp.float32),
    )


_combine_calls = (_make_combine(False), _make_combine(True))


def _prep_dir(p):
    scale = 1.0 / sqrt(_AD)
    qw = p['q_w'] * scale
    qb = p['q_b'] * scale
    kwx = p['k_w'][:_D]
    kwe = p['k_w'][_D:]
    k3 = kwe.reshape(_ED, _H, _AD)
    blocks = [k3[:, h, :].T for h in range(_H)]
    m = jax.scipy.linalg.block_diag(*blocks)
    wa = jnp.concatenate([qw, qw @ m], axis=1)
    ba = jnp.concatenate([qb, qb @ m])
    wb = jnp.concatenate([kwx, p['v_w']], axis=1)
    bb = jnp.concatenate([p['k_b'], p['v_b']])
    g1 = p['g_w'][:_OD, 0] + p['g_w'][2 * _OD:, 0]
    g2 = p['g_w'][_OD:2 * _OD, 0] - p['g_w'][2 * _OD:, 0]
    return wa, ba, wb, bb, jnp.concatenate([g1, g2]), p['g_b'][0]


def _prep_layer(lp):
    waf, baf, wbf, bbf, gvf, gbf = _prep_dir(lp['fw'])
    wab, bab, wbb, bbb, gvb, gbb = _prep_dir(lp['bw'])
    wcat = jnp.concatenate([waf, wbf, wab, wbb, lp['fw']['sl_w'],
                            lp['bw']['sl_w']], axis=1)
    bcat = jnp.concatenate([baf, bbf, bab, bbb, lp['fw']['sl_b'],
                            lp['bw']['sl_b']]).reshape(1, 640)
    gw = jnp.stack([gvf, gvb], axis=1)
    gb = jnp.stack([gbf, gbb]).reshape(1, 2)
    return dict(
        wcat=wcat, bcat=bcat, gw=gw, gb=gb,
        m1w=lp['m1_w'], m1b=lp['m1_b'].reshape(1, 2 * _D),
        m2w=lp['m2_w'], m2b=lp['m2_b'].reshape(1, _D),
        lng=lp['ln_g'].reshape(1, _D), lnb=lp['ln_b'].reshape(1, _D),
    )


def kernel(x, edge_index, edge_attr, params):
    f32 = jnp.float32
    i32 = jnp.int32
    x_p = jnp.zeros((_NP, _D), f32).at[:_N].set(x)
    sd_p = jnp.full((2, _EALLOC), _N, i32).at[:, :_E].set(
        edge_index.astype(i32)[::-1])
    ea_p = jnp.zeros((_EALLOC, _ED), f32).at[:_E].set(edge_attr)
    zrows = jnp.zeros((_NP, _AW), f32)

    outs = [x_p]
    for li, lp in enumerate(params['layers']):
        w = _prep_layer(lp)
        gafw, gbfw, gabw, gbbw, r2 = _proj_call(outs[-1], w['wcat'], w['bcat'])
        acc = _get_edge_call()(gafw, gbfw, gabw, gbbw, ea_p, sd_p, zrows)
        has_resid = li == 1
        resid = outs[1] if has_resid else outs[-1]
        y = _combine_calls[int(has_resid)](
            acc, r2, resid, w['gw'], w['gb'], w['m1w'], w['m1b'],
            w['m2w'], w['m2b'], w['lng'], w['lnb'])
        outs.append(y)
    return outs[-1][:_N]


# speedup vs baseline: 1.3340x; 1.2480x over previous
import functools
from math import sqrt

import jax
import jax.numpy as jnp
from jax import lax
from jax.experimental import pallas as pl
from jax.experimental.pallas import tpu as pltpu
from jax.experimental.pallas import tpu_sc as plsc

_N = 10000
_E = 320000
_D = 128
_ED = 16
_H = 4
_AD = 16
_OD = 64
_NP = 10240
_NT = 32
_C = 96
_NCH = 106
_EPT = _C * _NCH
_EPAD = _EPT * _NT
_EALLOC = _EPAD + _C
_RPT = _NP // 16
_BLK = 1280
_GRID = _NP // _BLK
_AW = 80


def _proj_body(x_ref, w_ref, b_ref, gafw, gbfw, gabw, gbbw, r2):
    y = jnp.dot(x_ref[...], w_ref[...], preferred_element_type=jnp.float32)
    y = y + b_ref[...]
    gafw[...] = y[:, 0:128].astype(jnp.bfloat16)
    gbfw[...] = y[:, 128:256].astype(jnp.bfloat16)
    gabw[...] = y[:, 256:384].astype(jnp.bfloat16)
    gbbw[...] = y[:, 384:512].astype(jnp.bfloat16)
    r2[...] = y[:, 512:640]


_proj_call = pl.pallas_call(
    _proj_body,
    grid=(_GRID,),
    in_specs=[
        pl.BlockSpec((_BLK, _D), lambda i: (i, 0)),
        pl.BlockSpec((_D, 640), lambda i: (0, 0)),
        pl.BlockSpec((1, 640), lambda i: (0, 0)),
    ],
    out_specs=[pl.BlockSpec((_BLK, _D), lambda i: (i, 0))] * 5,
    out_shape=[jax.ShapeDtypeStruct((_NP, _D), jnp.bfloat16)] * 4
    + [jax.ShapeDtypeStruct((_NP, _D), jnp.float32)],
)


def _edge_body(ga_fw, gb_fw, ga_bw, gb_bw, ea_p, sd_p, zrows, out,
               accs, idx0, idx1, bufa0, bufb0, bufe0, bufa1, bufb1, bufe1,
               msg, sa0, sb0, se0, sa1, sb1, se1):
    c = lax.axis_index("c")
    s = lax.axis_index("s")
    wid = s * 2 + c
    r0 = s * _RPT

    lane = lax.iota(jnp.int32, 16)
    ixs = [lane ^ 8, lane ^ 4, lane ^ 2, lane ^ 1]
    base_t = wid * _EPT
    sets = ((idx0, bufa0, bufb0, bufe0, sa0, sb0, se0),
            (idx1, bufa1, bufb1, bufe1, sa1, sb1, se1))

    def run_dir(d, ga_hbm, gb_hbm):
        agg_row = d
        oth_row = 1 - d
        pltpu.sync_copy(zrows.at[pl.ds(r0, _RPT)], accs.at[pl.ds(r0, _RPT)])
        plsc.subcore_barrier()

        def issue(g, st):
            idx, bufa, bufb, bufe, sa, sb, se = sets[st]
            base = base_t + g * _C
            pltpu.sync_copy(sd_p.at[:, pl.ds(base, _C)], idx)
            pltpu.async_copy(ga_hbm.at[idx.at[agg_row]], bufa, sa)
            pltpu.async_copy(gb_hbm.at[idx.at[oth_row]], bufb, sb)
            pltpu.async_copy(ea_p.at[pl.ds(base, _C)], bufe, se)

        def wait(st):
            idx, bufa, bufb, bufe, sa, sb, se = sets[st]
            pltpu.make_async_copy(ga_hbm.at[idx.at[agg_row]], bufa, sa).wait()
            pltpu.make_async_copy(gb_hbm.at[idx.at[oth_row]], bufb, sb).wait()
            pltpu.make_async_copy(ea_p.at[pl.ds(0, _C)], bufe, se).wait()

        def compute(st):
            idx, bufa, bufb, bufe, sa, sb, se = sets[st]

            def one_edge(i):
                ea_v = bufe[i, :]
                pf = plsc.PackFormat.INTERLEAVED
                qs = (*plsc.unpack(bufa[i, pl.ds(0, 32)], format=pf),
                      *plsc.unpack(bufa[i, pl.ds(32, 32)], format=pf))
                ts = (*plsc.unpack(bufa[i, pl.ds(64, 32)], format=pf),
                      *plsc.unpack(bufa[i, pl.ds(96, 32)], format=pf))
                ks = (*plsc.unpack(bufb[i, pl.ds(0, 32)], format=pf),
                      *plsc.unpack(bufb[i, pl.ds(32, 32)], format=pf))
                vs = (*plsc.unpack(bufb[i, pl.ds(64, 32)], format=pf),
                      *plsc.unpack(bufb[i, pl.ds(96, 32)], format=pf))
                wbrd = []
                for h in range(_H):
                    p = qs[h] * ks[h] + ea_v * ts[h]
                    for ix in ixs:
                        p = p + p.at[ix].get(mode='promise_in_bounds')
                    wbrd.append(jnp.exp(p))
                for h in range(_H):
                    msg[i, pl.ds(h * 16, 16)] = wbrd[h] * vs[h]
                w_pack = jnp.where(lane == 0, wbrd[0], wbrd[1])
                w_pack = jnp.where(lane == 2, wbrd[2], w_pack)
                w_pack = jnp.where(lane == 3, wbrd[3], w_pack)
                w_pack = jnp.where(lane < 4, w_pack, 0.0)
                msg[i, pl.ds(64, 16)] = w_pack

            def edge(j, ecarry):
                one_edge(2 * j)
                one_edge(2 * j + 1)
                return ecarry

            lax.fori_loop(0, _C // 2, edge, 0)
            pltpu.sync_copy(msg, accs.at[idx.at[agg_row]], add=True)

        issue(0, 0)

        def pair(k, carry):
            issue(2 * k + 1, 1)
            wait(0)
            compute(0)
            issue(2 * k + 2, 0)
            wait(1)
            compute(1)
            return carry

        lax.fori_loop(0, _NCH // 2, pair, 0)
        wait(0)
        plsc.subcore_barrier()
        pltpu.sync_copy(accs.at[pl.ds(r0, _RPT)],
                        out.at[c, d, pl.ds(r0, _RPT)])
        plsc.subcore_barrier()

    run_dir(0, ga_fw, gb_fw)
    run_dir(1, ga_bw, gb_bw)


@functools.cache
def _get_edge_call():
    return pl.kernel(
        _edge_body,
        out_type=jax.ShapeDtypeStruct((2, 2, _NP, _AW), jnp.float32),
        mesh=plsc.VectorSubcoreMesh(core_axis_name="c", subcore_axis_name="s"),
        scratch_types=[
            pltpu.VMEM_SHARED((_NP, _AW), jnp.float32),
            pltpu.VMEM((2, _C), jnp.int32),
            pltpu.VMEM((2, _C), jnp.int32),
            pltpu.VMEM((_C, _D), jnp.bfloat16),
            pltpu.VMEM((_C, _D), jnp.bfloat16),
            pltpu.VMEM((_C, _ED), jnp.float32),
            pltpu.VMEM((_C, _D), jnp.bfloat16),
            pltpu.VMEM((_C, _D), jnp.bfloat16),
            pltpu.VMEM((_C, _ED), jnp.float32),
            pltpu.VMEM((_C, _AW), jnp.float32),
            pltpu.SemaphoreType.DMA,
            pltpu.SemaphoreType.DMA,
            pltpu.SemaphoreType.DMA,
            pltpu.SemaphoreType.DMA,
            pltpu.SemaphoreType.DMA,
            pltpu.SemaphoreType.DMA,
        ],
        compiler_params=pltpu.CompilerParams(needs_layout_passes=False,
                                             use_tc_tiling_on_sc=False),
    )


def _combine_body(acc_ref, r2_ref, resid_ref, gw_ref, gb_ref,
                  m1w_ref, m1b_ref, m2w_ref, m2b_ref, lng_ref, lnb_ref,
                  o_ref, *, has_resid):
    acc = acc_ref[...]
    ys = []
    for dcol in range(2):
        a = acc[0, dcol] + acc[1, dcol]
        parts = [a[:, h * 16:(h + 1) * 16] / (a[:, 64 + h:65 + h] + 1e-16)
                 for h in range(_H)]
        outv = jnp.concatenate(parts, axis=1)
        res = r2_ref[:, dcol * 64:(dcol + 1) * 64]
        gin = jnp.dot(jnp.concatenate([outv, res], axis=1),
                      gw_ref[:, dcol:dcol + 1],
                      preferred_element_type=jnp.float32)
        gate = jax.nn.sigmoid(gin + gb_ref[0, dcol])
        ys.append(gate * outv + (1.0 - gate) * res)
    ret = jnp.concatenate(ys, axis=1)
    z = (jnp.dot(ret, m1w_ref[...], preferred_element_type=jnp.float32)
         + m1b_ref[...])
    h1 = 0.5 * z * (1.0 + lax.erf(z * (1.0 / sqrt(2.0))))
    y = jnp.dot(h1, m2w_ref[...], preferred_element_type=jnp.float32)
    y = y + m2b_ref[...] + ret
    mean = jnp.mean(y, axis=1, keepdims=True)
    var = jnp.mean((y - mean) ** 2, axis=1, keepdims=True)
    y = (y - mean) / jnp.sqrt(var + 1e-5) * lng_ref[...] + lnb_ref[...]
    if has_resid:
        y = y + resid_ref[...]
    o_ref[...] = y


def _make_combine(has_resid):
    return pl.pallas_call(
        functools.partial(_combine_body, has_resid=has_resid),
        grid=(_GRID,),
        in_specs=[
            pl.BlockSpec((2, 2, _BLK, _AW), lambda i: (0, 0, i, 0)),
            pl.BlockSpec((_BLK, _D), lambda i: (i, 0)),
            pl.BlockSpec((_BLK, _D), lambda i: (i, 0)),
            pl.BlockSpec((_D, 2), lambda i: (0, 0)),
            pl.BlockSpec((1, 2), lambda i: (0, 0)),
            pl.BlockSpec((_D, 2 * _D), lambda i: (0, 0)),
            pl.BlockSpec((1, 2 * _D), lambda i: (0, 0)),
            pl.BlockSpec((2 * _D, _D), lambda i: (0, 0)),
            pl.BlockSpec((1, _D), lambda i: (0, 0)),
            pl.BlockSpec((1, _D), lambda i: (0, 0)),
            pl.BlockSpec((1, _D), lambda i: (0, 0)),
        ],
        out_specs=pl.BlockSpec((_BLK, _D), lambda i: (i, 0)),
        out_shape=jax.ShapeDtypeStruct((_NP, _D), jnp.float32),
    )


_combine_calls = (_make_combine(False), _make_combine(True))


_ILV = [b + o for b in range(0, 128, 32)
        for j in range(16) for o in (j, 16 + j)]


def _prep_dir(p):
    scale = 1.0 / sqrt(_AD)
    qw = p['q_w'] * scale
    qb = p['q_b'] * scale
    kwx = p['k_w'][:_D]
    kwe = p['k_w'][_D:]
    k3 = kwe.reshape(_ED, _H, _AD)
    blocks = [k3[:, h, :].T for h in range(_H)]
    m = jax.scipy.linalg.block_diag(*blocks)
    ilv = jnp.array(_ILV)
    wa = jnp.concatenate([qw, qw @ m], axis=1)[:, ilv]
    ba = jnp.concatenate([qb, qb @ m])[ilv]
    wb = jnp.concatenate([kwx, p['v_w']], axis=1)[:, ilv]
    bb = jnp.concatenate([p['k_b'], p['v_b']])[ilv]
    g1 = p['g_w'][:_OD, 0] + p['g_w'][2 * _OD:, 0]
    g2 = p['g_w'][_OD:2 * _OD, 0] - p['g_w'][2 * _OD:, 0]
    return wa, ba, wb, bb, jnp.concatenate([g1, g2]), p['g_b'][0]


def _prep_layer(lp):
    waf, baf, wbf, bbf, gvf, gbf = _prep_dir(lp['fw'])
    wab, bab, wbb, bbb, gvb, gbb = _prep_dir(lp['bw'])
    wcat = jnp.concatenate([waf, wbf, wab, wbb, lp['fw']['sl_w'],
                            lp['bw']['sl_w']], axis=1)
    bcat = jnp.concatenate([baf, bbf, bab, bbb, lp['fw']['sl_b'],
                            lp['bw']['sl_b']]).reshape(1, 640)
    gw = jnp.stack([gvf, gvb], axis=1)
    gb = jnp.stack([gbf, gbb]).reshape(1, 2)
    return dict(
        wcat=wcat, bcat=bcat, gw=gw, gb=gb,
        m1w=lp['m1_w'], m1b=lp['m1_b'].reshape(1, 2 * _D),
        m2w=lp['m2_w'], m2b=lp['m2_b'].reshape(1, _D),
        lng=lp['ln_g'].reshape(1, _D), lnb=lp['ln_b'].reshape(1, _D),
    )


def kernel(x, edge_index, edge_attr, params):
    f32 = jnp.float32
    i32 = jnp.int32
    x_p = jnp.zeros((_NP, _D), f32).at[:_N].set(x)
    sd_p = jnp.full((2, _EALLOC), _N, i32).at[:, :_E].set(
        edge_index.astype(i32)[::-1])
    ea_p = jnp.zeros((_EALLOC, _ED), f32).at[:_E].set(edge_attr)
    zrows = jnp.zeros((_NP, _AW), f32)

    outs = [x_p]
    for li, lp in enumerate(params['layers']):
        w = _prep_layer(lp)
        gafw, gbfw, gabw, gbbw, r2 = _proj_call(outs[-1], w['wcat'], w['bcat'])
        acc = _get_edge_call()(gafw, gbfw, gabw, gbbw, ea_p, sd_p, zrows)
        has_resid = li == 1
        resid = outs[1] if has_resid else outs[-1]
        y = _combine_calls[int(has_resid)](
            acc, r2, resid, w['gw'], w['gb'], w['m1w'], w['m1b'],
            w['m2w'], w['m2b'], w['lng'], w['lnb'])
        outs.append(y)
    return outs[-1][:_N]


# speedup vs baseline: 1.3520x; 1.0135x over previous
import functools
from math import sqrt

import jax
import jax.numpy as jnp
from jax import lax
from jax.experimental import pallas as pl
from jax.experimental.pallas import tpu as pltpu
from jax.experimental.pallas import tpu_sc as plsc

_N = 10000
_E = 320000
_D = 128
_ED = 16
_H = 4
_AD = 16
_OD = 64
_NP = 10240
_NT = 32
_C = 128
_NCH = 80
_EPT = _C * _NCH
_EPAD = _EPT * _NT
_EALLOC = _EPAD + _C
_RPT = _NP // 16
_BLK = 1280
_GRID = _NP // _BLK
_AW = 80


def _proj_body(x_ref, w_ref, b_ref, gafw, gbfw, gabw, gbbw, r2):
    y = jnp.dot(x_ref[...], w_ref[...], preferred_element_type=jnp.float32)
    y = y + b_ref[...]
    gafw[...] = y[:, 0:128].astype(jnp.bfloat16)
    gbfw[...] = y[:, 128:256].astype(jnp.bfloat16)
    gabw[...] = y[:, 256:384].astype(jnp.bfloat16)
    gbbw[...] = y[:, 384:512].astype(jnp.bfloat16)
    r2[...] = y[:, 512:640]


_proj_call = pl.pallas_call(
    _proj_body,
    grid=(_GRID,),
    in_specs=[
        pl.BlockSpec((_BLK, _D), lambda i: (i, 0)),
        pl.BlockSpec((_D, 640), lambda i: (0, 0)),
        pl.BlockSpec((1, 640), lambda i: (0, 0)),
    ],
    out_specs=[pl.BlockSpec((_BLK, _D), lambda i: (i, 0))] * 5,
    out_shape=[jax.ShapeDtypeStruct((_NP, _D), jnp.bfloat16)] * 4
    + [jax.ShapeDtypeStruct((_NP, _D), jnp.float32)],
)


def _edge_body(ga_fw, gb_fw, ga_bw, gb_bw, ea_p, sd_p, zrows, out,
               accs, idx0, idx1, bufa0, bufb0, bufe0, bufa1, bufb1, bufe1,
               msg, sa0, sb0, se0, sa1, sb1, se1):
    c = lax.axis_index("c")
    s = lax.axis_index("s")
    wid = s * 2 + c
    r0 = s * _RPT

    lane = lax.iota(jnp.int32, 16)
    ixs = [lane ^ 8, lane ^ 4, lane ^ 2, lane ^ 1]
    base_t = wid * _EPT
    sets = ((idx0, bufa0, bufb0, bufe0, sa0, sb0, se0),
            (idx1, bufa1, bufb1, bufe1, sa1, sb1, se1))

    def run_dir(d, ga_hbm, gb_hbm):
        agg_row = d
        oth_row = 1 - d
        pltpu.sync_copy(zrows.at[pl.ds(r0, _RPT)], accs.at[pl.ds(r0, _RPT)])
        plsc.subcore_barrier()

        def issue(g, st):
            idx, bufa, bufb, bufe, sa, sb, se = sets[st]
            base = base_t + g * _C
            pltpu.sync_copy(sd_p.at[:, pl.ds(base, _C)], idx)
            pltpu.async_copy(ga_hbm.at[idx.at[agg_row]], bufa, sa)
            pltpu.async_copy(gb_hbm.at[idx.at[oth_row]], bufb, sb)
            pltpu.async_copy(ea_p.at[pl.ds(base, _C)], bufe, se)

        def wait(st):
            idx, bufa, bufb, bufe, sa, sb, se = sets[st]
            pltpu.make_async_copy(ga_hbm.at[idx.at[agg_row]], bufa, sa).wait()
            pltpu.make_async_copy(gb_hbm.at[idx.at[oth_row]], bufb, sb).wait()
            pltpu.make_async_copy(ea_p.at[pl.ds(0, _C)], bufe, se).wait()

        def compute(st):
            idx, bufa, bufb, bufe, sa, sb, se = sets[st]

            def one_edge(i):
                ea_v = bufe[i, :]
                pf = plsc.PackFormat.INTERLEAVED
                qs = (*plsc.unpack(bufa[i, pl.ds(0, 32)], format=pf),
                      *plsc.unpack(bufa[i, pl.ds(32, 32)], format=pf))
                ts = (*plsc.unpack(bufa[i, pl.ds(64, 32)], format=pf),
                      *plsc.unpack(bufa[i, pl.ds(96, 32)], format=pf))
                ks = (*plsc.unpack(bufb[i, pl.ds(0, 32)], format=pf),
                      *plsc.unpack(bufb[i, pl.ds(32, 32)], format=pf))
                vs = (*plsc.unpack(bufb[i, pl.ds(64, 32)], format=pf),
                      *plsc.unpack(bufb[i, pl.ds(96, 32)], format=pf))
                wbrd = []
                for h in range(_H):
                    p = qs[h] * ks[h] + ea_v * ts[h]
                    for ix in ixs:
                        p = p + p.at[ix].get(mode='promise_in_bounds')
                    wbrd.append(jnp.exp(p))
                for h in range(_H):
                    msg[i, pl.ds(h * 16, 16)] = wbrd[h] * vs[h]
                w_pack = jnp.where(lane == 0, wbrd[0], wbrd[1])
                w_pack = jnp.where(lane == 2, wbrd[2], w_pack)
                w_pack = jnp.where(lane == 3, wbrd[3], w_pack)
                w_pack = jnp.where(lane < 4, w_pack, 0.0)
                msg[i, pl.ds(64, 16)] = w_pack

            def edge(j, ecarry):
                one_edge(2 * j)
                one_edge(2 * j + 1)
                return ecarry

            lax.fori_loop(0, _C // 2, edge, 0)
            pltpu.sync_copy(msg, accs.at[idx.at[agg_row]], add=True)

        issue(0, 0)

        def pair(k, carry):
            issue(2 * k + 1, 1)
            wait(0)
            compute(0)
            issue(2 * k + 2, 0)
            wait(1)
            compute(1)
            return carry

        lax.fori_loop(0, _NCH // 2, pair, 0)
        wait(0)
        plsc.subcore_barrier()
        pltpu.sync_copy(accs.at[pl.ds(r0, _RPT)],
                        out.at[c, d, pl.ds(r0, _RPT)])
        plsc.subcore_barrier()

    run_dir(0, ga_fw, gb_fw)
    run_dir(1, ga_bw, gb_bw)


@functools.cache
def _get_edge_call():
    return pl.kernel(
        _edge_body,
        out_type=jax.ShapeDtypeStruct((2, 2, _NP, _AW), jnp.float32),
        mesh=plsc.VectorSubcoreMesh(core_axis_name="c", subcore_axis_name="s"),
        scratch_types=[
            pltpu.VMEM_SHARED((_NP, _AW), jnp.float32),
            pltpu.VMEM((2, _C), jnp.int32),
            pltpu.VMEM((2, _C), jnp.int32),
            pltpu.VMEM((_C, _D), jnp.bfloat16),
            pltpu.VMEM((_C, _D), jnp.bfloat16),
            pltpu.VMEM((_C, _ED), jnp.float32),
            pltpu.VMEM((_C, _D), jnp.bfloat16),
            pltpu.VMEM((_C, _D), jnp.bfloat16),
            pltpu.VMEM((_C, _ED), jnp.float32),
            pltpu.VMEM((_C, _AW), jnp.float32),
            pltpu.SemaphoreType.DMA,
            pltpu.SemaphoreType.DMA,
            pltpu.SemaphoreType.DMA,
            pltpu.SemaphoreType.DMA,
            pltpu.SemaphoreType.DMA,
            pltpu.SemaphoreType.DMA,
        ],
        compiler_params=pltpu.CompilerParams(needs_layout_passes=False,
                                             use_tc_tiling_on_sc=False),
    )


def _combine_body(acc_ref, r2_ref, resid_ref, gw_ref, gb_ref,
                  m1w_ref, m1b_ref, m2w_ref, m2b_ref, lng_ref, lnb_ref,
                  o_ref, *, has_resid):
    acc = acc_ref[...]
    ys = []
    for dcol in range(2):
        a = acc[0, dcol] + acc[1, dcol]
        parts = [a[:, h * 16:(h + 1) * 16] / (a[:, 64 + h:65 + h] + 1e-16)
                 for h in range(_H)]
        outv = jnp.concatenate(parts, axis=1)
        res = r2_ref[:, dcol * 64:(dcol + 1) * 64]
        gin = jnp.dot(jnp.concatenate([outv, res], axis=1),
                      gw_ref[:, dcol:dcol + 1],
                      preferred_element_type=jnp.float32)
        gate = jax.nn.sigmoid(gin + gb_ref[0, dcol])
        ys.append(gate * outv + (1.0 - gate) * res)
    ret = jnp.concatenate(ys, axis=1)
    z = (jnp.dot(ret, m1w_ref[...], preferred_element_type=jnp.float32)
         + m1b_ref[...])
    h1 = 0.5 * z * (1.0 + lax.erf(z * (1.0 / sqrt(2.0))))
    y = jnp.dot(h1, m2w_ref[...], preferred_element_type=jnp.float32)
    y = y + m2b_ref[...] + ret
    mean = jnp.mean(y, axis=1, keepdims=True)
    var = jnp.mean((y - mean) ** 2, axis=1, keepdims=True)
    y = (y - mean) / jnp.sqrt(var + 1e-5) * lng_ref[...] + lnb_ref[...]
    if has_resid:
        y = y + resid_ref[...]
    o_ref[...] = y


def _make_combine(has_resid):
    return pl.pallas_call(
        functools.partial(_combine_body, has_resid=has_resid),
        grid=(_GRID,),
        in_specs=[
            pl.BlockSpec((2, 2, _BLK, _AW), lambda i: (0, 0, i, 0)),
            pl.BlockSpec((_BLK, _D), lambda i: (i, 0)),
            pl.BlockSpec((_BLK, _D), lambda i: (i, 0)),
            pl.BlockSpec((_D, 2), lambda i: (0, 0)),
            pl.BlockSpec((1, 2), lambda i: (0, 0)),
            pl.BlockSpec((_D, 2 * _D), lambda i: (0, 0)),
            pl.BlockSpec((1, 2 * _D), lambda i: (0, 0)),
            pl.BlockSpec((2 * _D, _D), lambda i: (0, 0)),
            pl.BlockSpec((1, _D), lambda i: (0, 0)),
            pl.BlockSpec((1, _D), lambda i: (0, 0)),
            pl.BlockSpec((1, _D), lambda i: (0, 0)),
        ],
        out_specs=pl.BlockSpec((_BLK, _D), lambda i: (i, 0)),
        out_shape=jax.ShapeDtypeStruct((_NP, _D), jnp.float32),
    )


_combine_calls = (_make_combine(False), _make_combine(True))


_ILV = [b + o for b in range(0, 128, 32)
        for j in range(16) for o in (j, 16 + j)]


def _prep_dir(p):
    scale = 1.0 / sqrt(_AD)
    qw = p['q_w'] * scale
    qb = p['q_b'] * scale
    kwx = p['k_w'][:_D]
    kwe = p['k_w'][_D:]
    k3 = kwe.reshape(_ED, _H, _AD)
    blocks = [k3[:, h, :].T for h in range(_H)]
    m = jax.scipy.linalg.block_diag(*blocks)
    ilv = jnp.array(_ILV)
    wa = jnp.concatenate([qw, qw @ m], axis=1)[:, ilv]
    ba = jnp.concatenate([qb, qb @ m])[ilv]
    wb = jnp.concatenate([kwx, p['v_w']], axis=1)[:, ilv]
    bb = jnp.concatenate([p['k_b'], p['v_b']])[ilv]
    g1 = p['g_w'][:_OD, 0] + p['g_w'][2 * _OD:, 0]
    g2 = p['g_w'][_OD:2 * _OD, 0] - p['g_w'][2 * _OD:, 0]
    return wa, ba, wb, bb, jnp.concatenate([g1, g2]), p['g_b'][0]


def _prep_layer(lp):
    waf, baf, wbf, bbf, gvf, gbf = _prep_dir(lp['fw'])
    wab, bab, wbb, bbb, gvb, gbb = _prep_dir(lp['bw'])
    wcat = jnp.concatenate([waf, wbf, wab, wbb, lp['fw']['sl_w'],
                            lp['bw']['sl_w']], axis=1)
    bcat = jnp.concatenate([baf, bbf, bab, bbb, lp['fw']['sl_b'],
                            lp['bw']['sl_b']]).reshape(1, 640)
    gw = jnp.stack([gvf, gvb], axis=1)
    gb = jnp.stack([gbf, gbb]).reshape(1, 2)
    return dict(
        wcat=wcat, bcat=bcat, gw=gw, gb=gb,
        m1w=lp['m1_w'], m1b=lp['m1_b'].reshape(1, 2 * _D),
        m2w=lp['m2_w'], m2b=lp['m2_b'].reshape(1, _D),
        lng=lp['ln_g'].reshape(1, _D), lnb=lp['ln_b'].reshape(1, _D),
    )


def kernel(x, edge_index, edge_attr, params):
    f32 = jnp.float32
    i32 = jnp.int32
    x_p = jnp.zeros((_NP, _D), f32).at[:_N].set(x)
    sd_p = jnp.full((2, _EALLOC), _N, i32).at[:, :_E].set(
        edge_index.astype(i32)[::-1])
    ea_p = jnp.zeros((_EALLOC, _ED), f32).at[:_E].set(edge_attr)
    zrows = jnp.zeros((_NP, _AW), f32)

    outs = [x_p]
    for li, lp in enumerate(params['layers']):
        w = _prep_layer(lp)
        gafw, gbfw, gabw, gbbw, r2 = _proj_call(outs[-1], w['wcat'], w['bcat'])
        acc = _get_edge_call()(gafw, gbfw, gabw, gbbw, ea_p, sd_p, zrows)
        has_resid = li == 1
        resid = outs[1] if has_resid else outs[-1]
        y = _combine_calls[int(has_resid)](
            acc, r2, resid, w['gw'], w['gb'], w['m1w'], w['m1b'],
            w['m2w'], w['m2b'], w['lng'], w['lnb'])
        outs.append(y)
    return outs[-1][:_N]
